# baseline (speedup 1.0000x reference)
import functools

import jax
import jax.numpy as jnp
from jax.experimental import pallas as pl
from jax.experimental.pallas import tpu as pltpu

B, N, NODE_DIM = 4, 512, 128
EMB = 64
NH = 2

_PREC = None


def _dot_bf16x3(a, b):
    a_hi = a.astype(jnp.bfloat16)
    a_lo = (a - a_hi.astype(jnp.float32)).astype(jnp.bfloat16)
    b_hi = b.astype(jnp.bfloat16)
    b_lo = (b - b_hi.astype(jnp.float32)).astype(jnp.bfloat16)
    d = lambda x, y: jnp.dot(x, y, preferred_element_type=jnp.float32)
    return d(a_hi, b_hi) + (d(a_hi, b_lo) + d(a_lo, b_hi))


def _gat_layer(h, mask_t, w_ref, a_ref, b_ref, concat):
    proj = jnp.dot(h, w_ref[:], preferred_element_type=jnp.float32,
                   precision=_PREC)
    ps = proj * a_ref[0:1, :]
    pd = proj * a_ref[1:2, :]
    ssrc0 = jnp.sum(ps[:, :EMB], axis=1)
    ssrc1 = jnp.sum(ps[:, EMB:], axis=1)
    sdst0 = jnp.sum(pd[:, :EMB], axis=1)
    sdst1 = jnp.sum(pd[:, EMB:], axis=1)
    e0 = sdst0[:, None] + ssrc0[None, :]
    e1 = sdst1[:, None] + ssrc1[None, :]
    e0 = jnp.maximum(e0, 0.2 * e0)
    e1 = jnp.maximum(e1, 0.2 * e1)
    neg = jnp.float32(-jnp.inf)
    e0 = jnp.where(mask_t, e0, neg)
    e1 = jnp.where(mask_t, e1, neg)
    m = jnp.maximum(jnp.max(e0), jnp.max(e1))
    ex0 = jnp.exp(e0 - m)
    ex1 = jnp.exp(e1 - m)
    r0 = 1.0 / (jnp.sum(ex0, axis=1) + 1e-16)
    r1 = 1.0 / (jnp.sum(ex1, axis=1) + 1e-16)
    if concat:
        hp = jax.lax.Precision.HIGHEST
        u0 = jnp.dot(ex0, proj[:, :EMB], preferred_element_type=jnp.float32,
                     precision=hp)
        u1 = jnp.dot(ex1, proj[:, EMB:], preferred_element_type=jnp.float32,
                     precision=hp)
    else:
        u0 = _dot_bf16x3(ex0, proj[:, :EMB])
        u1 = _dot_bf16x3(ex1, proj[:, EMB:])
    o0 = u0 * r0[:, None]
    o1 = u1 * r1[:, None]
    if concat:
        o = jnp.concatenate([o0, o1], axis=1) + b_ref[0:1, :]
        return jnp.where(o > 0.0, o, jnp.exp(jnp.minimum(o, 0.0)) - 1.0)
    return 0.5 * (o0 + o1) + b_ref[0:1, :]


def _qnet_kernel(xv_ref, wst_ref, w0_ref, a0_ref, b0_ref, w1_ref, a1_ref,
                 b1_ref, t6w_ref, t6b_ref, t7w_ref, t7b_ref, t5_ref,
                 t5b_ref, out_ref):
    h = xv_ref[0]
    mask_t = wst_ref[0] != 0

    h1 = _gat_layer(h, mask_t, w0_ref, a0_ref, b0_ref, True)
    mu = _gat_layer(h1, mask_t, w1_ref, a1_ref, b1_ref, False)

    musum = jnp.sum(mu, axis=0, keepdims=True)
    gs = jnp.dot(musum, t6w_ref[:], preferred_element_type=jnp.float32,
                 precision=_PREC) + t6b_ref[0:1, :]
    la = jnp.dot(mu, t7w_ref[:], preferred_element_type=jnp.float32,
                 precision=_PREC) + t7b_ref[0:1, :]
    cat = jnp.concatenate([jnp.broadcast_to(gs, (N, EMB)), la], axis=1)
    cat = jnp.maximum(cat, 0.0)
    vals = jnp.dot(cat, t5_ref[:], preferred_element_type=jnp.float32,
                   precision=_PREC)
    out_ref[0] = vals[:, 0][None, :] + t5b_ref[0, 0]


@functools.partial(jax.jit, static_argnames=())
def _run(xv, WsT, W0, a0, b0, W1, a1, b1, t6w, t6b, t7w, t7b, t5, t5b):
    full = lambda shape: pl.BlockSpec(shape, lambda i: tuple(0 for _ in shape))
    out = pl.pallas_call(
        _qnet_kernel,
        grid=(B,),
        in_specs=[
            pl.BlockSpec((1, N, NODE_DIM), lambda i: (i, 0, 0)),
            pl.BlockSpec((1, N, N), lambda i: (i, 0, 0)),
            full((NODE_DIM, NH * EMB)),
            full((2, NH * EMB)),
            full((1, NH * EMB)),
            full((NH * EMB, NH * EMB)),
            full((2, NH * EMB)),
            full((1, EMB)),
            full((EMB, EMB)),
            full((1, EMB)),
            full((EMB, EMB)),
            full((1, EMB)),
            full((NH * EMB, 1)),
            full((1, 1)),
        ],
        out_specs=pl.BlockSpec((1, 1, N), lambda i: (i, 0, 0)),
        out_shape=jax.ShapeDtypeStruct((B, 1, N), jnp.float32),
        compiler_params=pltpu.CompilerParams(
            dimension_semantics=("parallel",)),
    )(xv, WsT, W0, a0, b0, W1, a1, b1, t6w, t6b, t7w, t7b, t5, t5b)
    return out.reshape(B, N)


def kernel(xv, Ws, pyg_data, W0, asrc0, adst0, b0, W1, asrc1, adst1, b1,
           t5_w, t5_b, t6_w, t6_b, t7_w, t7_b):
    a0 = jnp.concatenate([asrc0.reshape(1, NH * EMB),
                          adst0.reshape(1, NH * EMB)], axis=0)
    a1 = jnp.concatenate([asrc1.reshape(1, NH * EMB),
                          adst1.reshape(1, NH * EMB)], axis=0)
    return _run(xv, Ws.transpose(0, 2, 1), W0, a0, b0.reshape(1, NH * EMB),
                W1, a1, b1.reshape(1, EMB), t6_w, t6_b.reshape(1, EMB), t7_w,
                t7_b.reshape(1, EMB), t5_w, t5_b.reshape(1, 1))


# speedup vs baseline: 1.0440x; 1.0440x over previous
import functools

import jax
import jax.numpy as jnp
from jax.experimental import pallas as pl
from jax.experimental.pallas import tpu as pltpu

B, N, NODE_DIM = 4, 512, 128
EMB = 64
NH = 2

_PREC = None


def _dot_bf16x3(a, b):
    a_hi = a.astype(jnp.bfloat16)
    a_lo = (a - a_hi.astype(jnp.float32)).astype(jnp.bfloat16)
    b_hi = b.astype(jnp.bfloat16)
    b_lo = (b - b_hi.astype(jnp.float32)).astype(jnp.bfloat16)
    d = lambda x, y: jnp.dot(x, y, preferred_element_type=jnp.float32)
    return d(a_hi, b_hi) + (d(a_hi, b_lo) + d(a_lo, b_hi))


def _gat_layer(h, mask_t, w_ref, a_ref, b_ref, concat):
    proj = jnp.dot(h, w_ref[:], preferred_element_type=jnp.float32,
                   precision=_PREC)
    ps = proj * a_ref[0:1, :]
    pd = proj * a_ref[1:2, :]
    ssrc0 = jnp.sum(ps[:, :EMB], axis=1)
    ssrc1 = jnp.sum(ps[:, EMB:], axis=1)
    sdst0 = jnp.sum(pd[:, :EMB], axis=1)
    sdst1 = jnp.sum(pd[:, EMB:], axis=1)
    e0 = sdst0[:, None] + ssrc0[None, :]
    e1 = sdst1[:, None] + ssrc1[None, :]
    e0 = jnp.maximum(e0, 0.2 * e0)
    e1 = jnp.maximum(e1, 0.2 * e1)
    neg = jnp.float32(-jnp.inf)
    e0 = jnp.where(mask_t, e0, neg)
    e1 = jnp.where(mask_t, e1, neg)
    m = jnp.maximum(jnp.max(e0), jnp.max(e1))
    ex0 = jnp.exp(e0 - m)
    ex1 = jnp.exp(e1 - m)
    r0 = 1.0 / (jnp.sum(ex0, axis=1) + 1e-16)
    r1 = 1.0 / (jnp.sum(ex1, axis=1) + 1e-16)
    if concat:
        hp = jax.lax.Precision.HIGHEST
        u0 = jnp.dot(ex0, proj[:, :EMB], preferred_element_type=jnp.float32,
                     precision=hp)
        u1 = jnp.dot(ex1, proj[:, EMB:], preferred_element_type=jnp.float32,
                     precision=hp)
    else:
        u0 = _dot_bf16x3(ex0, proj[:, :EMB])
        u1 = _dot_bf16x3(ex1, proj[:, EMB:])
    o0 = u0 * r0[:, None]
    o1 = u1 * r1[:, None]
    if concat:
        o = jnp.concatenate([o0, o1], axis=1) + b_ref[0:1, :]
        return jnp.where(o > 0.0, o, jnp.exp(jnp.minimum(o, 0.0)) - 1.0)
    return 0.5 * (o0 + o1) + b_ref[0:1, :]


def _qnet_kernel(xv_ref, wst_ref, w0_ref, a0_ref, b0_ref, w1_ref, a1_ref,
                 b1_ref, t6w_ref, t6b_ref, t7w_ref, t7b_ref, t5_ref,
                 t5b_ref, out_ref):
    rows = []
    for i in range(B):
        h = xv_ref[i]
        mask_t = wst_ref[i] != 0

        h1 = _gat_layer(h, mask_t, w0_ref, a0_ref, b0_ref, True)
        mu = _gat_layer(h1, mask_t, w1_ref, a1_ref, b1_ref, False)

        musum = jnp.sum(mu, axis=0, keepdims=True)
        gs = jnp.dot(musum, t6w_ref[:], preferred_element_type=jnp.float32,
                     precision=_PREC) + t6b_ref[0:1, :]
        la = jnp.dot(mu, t7w_ref[:], preferred_element_type=jnp.float32,
                     precision=_PREC) + t7b_ref[0:1, :]
        cat = jnp.concatenate([jnp.broadcast_to(gs, (N, EMB)), la], axis=1)
        cat = jnp.maximum(cat, 0.0)
        vals = jnp.dot(cat, t5_ref[:], preferred_element_type=jnp.float32,
                       precision=_PREC)
        rows.append(vals[:, 0][None, :] + t5b_ref[0, 0])
    out_ref[...] = jnp.concatenate(rows, axis=0)


@functools.partial(jax.jit, static_argnames=())
def _run(xv, WsT, W0, a0, b0, W1, a1, b1, t6w, t6b, t7w, t7b, t5, t5b):
    out = pl.pallas_call(
        _qnet_kernel,
        out_shape=jax.ShapeDtypeStruct((B, N), jnp.float32),
    )(xv, WsT, W0, a0, b0, W1, a1, b1, t6w, t6b, t7w, t7b, t5, t5b)
    return out


def kernel(xv, Ws, pyg_data, W0, asrc0, adst0, b0, W1, asrc1, adst1, b1,
           t5_w, t5_b, t6_w, t6_b, t7_w, t7_b):
    a0 = jnp.concatenate([asrc0.reshape(1, NH * EMB),
                          adst0.reshape(1, NH * EMB)], axis=0)
    a1 = jnp.concatenate([asrc1.reshape(1, NH * EMB),
                          adst1.reshape(1, NH * EMB)], axis=0)
    return _run(xv, Ws.transpose(0, 2, 1), W0, a0, b0.reshape(1, NH * EMB),
                W1, a1, b1.reshape(1, EMB), t6_w, t6_b.reshape(1, EMB), t7_w,
                t7_b.reshape(1, EMB), t5_w, t5_b.reshape(1, 1))


# speedup vs baseline: 1.1601x; 1.1112x over previous
import functools

import jax
import jax.numpy as jnp
from jax.experimental import pallas as pl
from jax.experimental.pallas import tpu as pltpu

B, N, NODE_DIM = 4, 512, 128
EMB = 64
NH = 2

_PREC = None


def _dot_bf16x3(a, b):
    a_hi = a.astype(jnp.bfloat16)
    a_lo = (a - a_hi.astype(jnp.float32)).astype(jnp.bfloat16)
    b_hi = b.astype(jnp.bfloat16)
    b_lo = (b - b_hi.astype(jnp.float32)).astype(jnp.bfloat16)
    d = lambda x, y: jnp.dot(x, y, preferred_element_type=jnp.float32)
    return d(a_hi, b_hi) + (d(a_hi, b_lo) + d(a_lo, b_hi))


def _gat_layer(h, mask_t, w_ref, a_ref, b_ref, concat):
    proj = jnp.dot(h, w_ref[:], preferred_element_type=jnp.float32,
                   precision=_PREC)
    ps = proj * a_ref[0:1, :]
    pd = proj * a_ref[1:2, :]
    ssrc0 = jnp.sum(ps[:, :EMB], axis=1)
    ssrc1 = jnp.sum(ps[:, EMB:], axis=1)
    sdst0 = jnp.sum(pd[:, :EMB], axis=1)
    sdst1 = jnp.sum(pd[:, EMB:], axis=1)
    e0 = sdst0[:, None] + ssrc0[None, :]
    e1 = sdst1[:, None] + ssrc1[None, :]
    e0 = jnp.maximum(e0, 0.2 * e0)
    e1 = jnp.maximum(e1, 0.2 * e1)
    neg = jnp.float32(-jnp.inf)
    e0 = jnp.where(mask_t, e0, neg)
    e1 = jnp.where(mask_t, e1, neg)
    m = jnp.maximum(jnp.max(e0), jnp.max(e1))
    ex0 = jnp.exp(e0 - m)
    ex1 = jnp.exp(e1 - m)
    r0 = 1.0 / (jnp.sum(ex0, axis=1) + 1e-16)
    r1 = 1.0 / (jnp.sum(ex1, axis=1) + 1e-16)
    if concat:
        hp = jax.lax.Precision.HIGHEST
        u0 = jnp.dot(ex0, proj[:, :EMB], preferred_element_type=jnp.float32,
                     precision=hp)
        u1 = jnp.dot(ex1, proj[:, EMB:], preferred_element_type=jnp.float32,
                     precision=hp)
    else:
        u0 = _dot_bf16x3(ex0, proj[:, :EMB])
        u1 = _dot_bf16x3(ex1, proj[:, EMB:])
    o0 = u0 * r0[:, None]
    o1 = u1 * r1[:, None]
    if concat:
        o = jnp.concatenate([o0, o1], axis=1) + b_ref[0:1, :]
        return jnp.where(o > 0.0, o, jnp.exp(jnp.minimum(o, 0.0)) - 1.0)
    return 0.5 * (o0 + o1) + b_ref[0:1, :]


def _qnet_kernel(xv_ref, wst_ref, w0_ref, a0_ref, b0_ref, w1_ref, a1_ref,
                 b1_ref, t6w_ref, t6b_ref, t7w_ref, t7b_ref, t5_ref,
                 t5b_ref, out_ref):
    rows = []
    for i in range(B):
        h = xv_ref[i]
        mask_t = wst_ref[i].T != 0

        h1 = _gat_layer(h, mask_t, w0_ref, a0_ref, b0_ref, True)
        mu = _gat_layer(h1, mask_t, w1_ref, a1_ref, b1_ref, False)

        musum = jnp.sum(mu, axis=0, keepdims=True)
        gs = jnp.dot(musum, t6w_ref[:], preferred_element_type=jnp.float32,
                     precision=_PREC) + t6b_ref[0:1, :]
        la = jnp.dot(mu, t7w_ref[:], preferred_element_type=jnp.float32,
                     precision=_PREC) + t7b_ref[0:1, :]
        cat = jnp.concatenate([jnp.broadcast_to(gs, (N, EMB)), la], axis=1)
        cat = jnp.maximum(cat, 0.0)
        vals = jnp.dot(cat, t5_ref[:], preferred_element_type=jnp.float32,
                       precision=_PREC)
        rows.append(vals[:, 0][None, :] + t5b_ref[0, 0])
    out_ref[...] = jnp.concatenate(rows, axis=0)


@functools.partial(jax.jit, static_argnames=())
def _run(xv, WsT, W0, a0, b0, W1, a1, b1, t6w, t6b, t7w, t7b, t5, t5b):
    out = pl.pallas_call(
        _qnet_kernel,
        out_shape=jax.ShapeDtypeStruct((B, N), jnp.float32),
    )(xv, WsT, W0, a0, b0, W1, a1, b1, t6w, t6b, t7w, t7b, t5, t5b)
    return out


def kernel(xv, Ws, pyg_data, W0, asrc0, adst0, b0, W1, asrc1, adst1, b1,
           t5_w, t5_b, t6_w, t6_b, t7_w, t7_b):
    a0 = jnp.concatenate([asrc0.reshape(1, NH * EMB),
                          adst0.reshape(1, NH * EMB)], axis=0)
    a1 = jnp.concatenate([asrc1.reshape(1, NH * EMB),
                          adst1.reshape(1, NH * EMB)], axis=0)
    return _run(xv, Ws, W0, a0, b0.reshape(1, NH * EMB),
                W1, a1, b1.reshape(1, EMB), t6_w, t6_b.reshape(1, EMB), t7_w,
                t7_b.reshape(1, EMB), t5_w, t5_b.reshape(1, 1))


# speedup vs baseline: 1.2099x; 1.0429x over previous
import functools

import jax
import jax.numpy as jnp
from jax.experimental import pallas as pl
from jax.experimental.pallas import tpu as pltpu

B, N, NODE_DIM = 4, 512, 128
EMB = 64
NH = 2

_PREC = None


def _dot_bf16x3(a, b):
    a_hi = a.astype(jnp.bfloat16)
    a_lo = (a - a_hi.astype(jnp.float32)).astype(jnp.bfloat16)
    b_hi = b.astype(jnp.bfloat16)
    b_lo = (b - b_hi.astype(jnp.float32)).astype(jnp.bfloat16)
    d = lambda x, y: jnp.dot(x, y, preferred_element_type=jnp.float32)
    return d(a_hi, b_hi) + (d(a_hi, b_lo) + d(a_lo, b_hi))


def _gat_layer(h, mask_t, w_ref, a_ref, b_ref, concat):
    proj = jnp.dot(h, w_ref[:], preferred_element_type=jnp.float32,
                   precision=_PREC)
    s = jnp.dot(proj, a_ref[:], preferred_element_type=jnp.float32,
                precision=jax.lax.Precision.HIGHEST)
    ssrc0 = s[:, 0]
    ssrc1 = s[:, 1]
    sdst0 = s[:, 2]
    sdst1 = s[:, 3]
    e0 = sdst0[:, None] + ssrc0[None, :]
    e1 = sdst1[:, None] + ssrc1[None, :]
    e0 = jnp.maximum(e0, 0.2 * e0)
    e1 = jnp.maximum(e1, 0.2 * e1)
    neg = jnp.float32(-jnp.inf)
    e0 = jnp.where(mask_t, e0, neg)
    e1 = jnp.where(mask_t, e1, neg)
    m = jnp.maximum(jnp.max(e0), jnp.max(e1))
    ex0 = jnp.exp(e0 - m)
    ex1 = jnp.exp(e1 - m)
    r0 = 1.0 / (jnp.sum(ex0, axis=1) + 1e-16)
    r1 = 1.0 / (jnp.sum(ex1, axis=1) + 1e-16)
    if concat:
        hp = jax.lax.Precision.HIGHEST
        u0 = jnp.dot(ex0, proj[:, :EMB], preferred_element_type=jnp.float32,
                     precision=hp)
        u1 = jnp.dot(ex1, proj[:, EMB:], preferred_element_type=jnp.float32,
                     precision=hp)
    else:
        u0 = _dot_bf16x3(ex0, proj[:, :EMB])
        u1 = _dot_bf16x3(ex1, proj[:, EMB:])
    o0 = u0 * r0[:, None]
    o1 = u1 * r1[:, None]
    if concat:
        o = jnp.concatenate([o0, o1], axis=1) + b_ref[0:1, :]
        return jnp.where(o > 0.0, o, jnp.exp(jnp.minimum(o, 0.0)) - 1.0)
    return 0.5 * (o0 + o1) + b_ref[0:1, :]


def _qnet_kernel(xv_ref, wst_ref, w0_ref, a0_ref, b0_ref, w1_ref, a1_ref,
                 b1_ref, t6w_ref, t6b_ref, t7w_ref, t7b_ref, t5_ref,
                 t5b_ref, out_ref):
    rows = []
    for i in range(B):
        h = xv_ref[i]
        mask_t = wst_ref[i].T != 0

        h1 = _gat_layer(h, mask_t, w0_ref, a0_ref, b0_ref, True)
        mu = _gat_layer(h1, mask_t, w1_ref, a1_ref, b1_ref, False)

        musum = jnp.sum(mu, axis=0, keepdims=True)
        gs = jnp.dot(musum, t6w_ref[:], preferred_element_type=jnp.float32,
                     precision=_PREC) + t6b_ref[0:1, :]
        la = jnp.dot(mu, t7w_ref[:], preferred_element_type=jnp.float32,
                     precision=_PREC) + t7b_ref[0:1, :]
        cat = jnp.concatenate([jnp.broadcast_to(gs, (N, EMB)), la], axis=1)
        cat = jnp.maximum(cat, 0.0)
        vals = jnp.dot(cat, t5_ref[:], preferred_element_type=jnp.float32,
                       precision=_PREC)
        rows.append(vals[:, 0][None, :] + t5b_ref[0, 0])
    out_ref[...] = jnp.concatenate(rows, axis=0)


@functools.partial(jax.jit, static_argnames=())
def _run(xv, WsT, W0, a0, b0, W1, a1, b1, t6w, t6b, t7w, t7b, t5, t5b):
    out = pl.pallas_call(
        _qnet_kernel,
        out_shape=jax.ShapeDtypeStruct((B, N), jnp.float32),
    )(xv, WsT, W0, a0, b0, W1, a1, b1, t6w, t6b, t7w, t7b, t5, t5b)
    return out


def _amat(asrc, adst):
    z = jnp.zeros((EMB,), jnp.float32)
    c0 = jnp.concatenate([asrc[0], z])
    c1 = jnp.concatenate([z, asrc[1]])
    c2 = jnp.concatenate([adst[0], z])
    c3 = jnp.concatenate([z, adst[1]])
    return jnp.stack([c0, c1, c2, c3], axis=1)


def kernel(xv, Ws, pyg_data, W0, asrc0, adst0, b0, W1, asrc1, adst1, b1,
           t5_w, t5_b, t6_w, t6_b, t7_w, t7_b):
    a0 = _amat(asrc0, adst0)
    a1 = _amat(asrc1, adst1)
    return _run(xv, Ws, W0, a0, b0.reshape(1, NH * EMB),
                W1, a1, b1.reshape(1, EMB), t6_w, t6_b.reshape(1, EMB), t7_w,
                t7_b.reshape(1, EMB), t5_w, t5_b.reshape(1, 1))


# speedup vs baseline: 1.2160x; 1.0050x over previous
import functools

import jax
import jax.numpy as jnp
from jax.experimental import pallas as pl

B, N, NODE_DIM = 4, 512, 128
EMB = 64
NH = 2

_PREC = None


def _dot_bf16x3(a, b):
    a_hi = a.astype(jnp.bfloat16)
    a_lo = (a - a_hi.astype(jnp.float32)).astype(jnp.bfloat16)
    b_hi = b.astype(jnp.bfloat16)
    b_lo = (b - b_hi.astype(jnp.float32)).astype(jnp.bfloat16)
    d = lambda x, y: jnp.dot(x, y, preferred_element_type=jnp.float32)
    return d(a_hi, b_hi) + (d(a_hi, b_lo) + d(a_lo, b_hi))


def _gat_layer(h, mask_t, w_ref, a_ref, b_ref, concat):
    proj = jnp.dot(h, w_ref[:], preferred_element_type=jnp.float32,
                   precision=_PREC)
    s = jnp.dot(proj, a_ref[:], preferred_element_type=jnp.float32,
                precision=jax.lax.Precision.HIGHEST)
    ssrc0 = s[:, 0]
    ssrc1 = s[:, 1]
    sdst0 = s[:, 2]
    sdst1 = s[:, 3]
    e0 = sdst0[:, None] + ssrc0[None, :]
    e1 = sdst1[:, None] + ssrc1[None, :]
    e0 = jnp.maximum(e0, 0.2 * e0)
    e1 = jnp.maximum(e1, 0.2 * e1)
    neg = jnp.float32(-jnp.inf)
    e0 = jnp.where(mask_t, e0, neg)
    e1 = jnp.where(mask_t, e1, neg)
    m = jnp.maximum(jnp.max(e0), jnp.max(e1))
    ex0 = jnp.exp(e0 - m)
    ex1 = jnp.exp(e1 - m)
    r0 = 1.0 / (jnp.sum(ex0, axis=1) + 1e-16)
    r1 = 1.0 / (jnp.sum(ex1, axis=1) + 1e-16)
    if concat:
        hp = jax.lax.Precision.HIGHEST
        u0 = jnp.dot(ex0, proj[:, :EMB], preferred_element_type=jnp.float32,
                     precision=hp)
        u1 = jnp.dot(ex1, proj[:, EMB:], preferred_element_type=jnp.float32,
                     precision=hp)
    else:
        u0 = _dot_bf16x3(ex0, proj[:, :EMB])
        u1 = _dot_bf16x3(ex1, proj[:, EMB:])
    o0 = u0 * r0[:, None]
    o1 = u1 * r1[:, None]
    if concat:
        o = jnp.concatenate([o0, o1], axis=1) + b_ref[0:1, :]
        return jnp.where(o > 0.0, o, jnp.exp(jnp.minimum(o, 0.0)) - 1.0)
    return 0.5 * (o0 + o1) + b_ref[0:1, :]


def _qnet_kernel(xv_ref, ws_ref, w0_ref, a0_ref, b0_ref, w1_ref, a1_ref,
                 b1_ref, t6w_ref, t6b_ref, t7w_ref, t7b_ref, t5_ref,
                 t5b_ref, out_ref):
    rows = []
    for i in range(B):
        h = xv_ref[i]
        mask_t = ws_ref[i].T != 0

        h1 = _gat_layer(h, mask_t, w0_ref, a0_ref, b0_ref, True)
        mu = _gat_layer(h1, mask_t, w1_ref, a1_ref, b1_ref, False)

        musum = jnp.sum(mu, axis=0, keepdims=True)
        gs = jnp.dot(musum, t6w_ref[:], preferred_element_type=jnp.float32,
                     precision=_PREC) + t6b_ref[0:1, :]
        la = jnp.dot(mu, t7w_ref[:], preferred_element_type=jnp.float32,
                     precision=_PREC) + t7b_ref[0:1, :]
        cat = jnp.concatenate([jnp.broadcast_to(gs, (N, EMB)), la], axis=1)
        cat = jnp.maximum(cat, 0.0)
        vals = jnp.dot(cat, t5_ref[:], preferred_element_type=jnp.float32,
                       precision=_PREC)
        rows.append(vals[:, 0][None, :] + t5b_ref[0, 0])
    out_ref[...] = jnp.concatenate(rows, axis=0)


@functools.partial(jax.jit, static_argnames=())
def _run(xv, Ws, W0, a0, b0, W1, a1, b1, t6w, t6b, t7w, t7b, t5, t5b):
    out = pl.pallas_call(
        _qnet_kernel,
        out_shape=jax.ShapeDtypeStruct((B, N), jnp.float32),
    )(xv, Ws, W0, a0, b0, W1, a1, b1, t6w, t6b, t7w, t7b, t5, t5b)
    return out


def _amat(asrc, adst):
    z = jnp.zeros((EMB,), jnp.float32)
    c0 = jnp.concatenate([asrc[0], z])
    c1 = jnp.concatenate([z, asrc[1]])
    c2 = jnp.concatenate([adst[0], z])
    c3 = jnp.concatenate([z, adst[1]])
    return jnp.stack([c0, c1, c2, c3], axis=1)


def kernel(xv, Ws, pyg_data, W0, asrc0, adst0, b0, W1, asrc1, adst1, b1,
           t5_w, t5_b, t6_w, t6_b, t7_w, t7_b):
    a0 = _amat(asrc0, adst0)
    a1 = _amat(asrc1, adst1)
    return _run(xv, Ws, W0, a0, b0.reshape(1, NH * EMB),
                W1, a1, b1.reshape(1, EMB), t6_w, t6_b.reshape(1, EMB), t7_w,
                t7_b.reshape(1, EMB), t5_w, t5_b.reshape(1, 1))
